# baseline (speedup 1.0000x reference)
import functools

import jax
import jax.numpy as jnp
from jax import lax
from jax.experimental import pallas as pl
from jax.experimental.pallas import tpu as pltpu
from jax.experimental.pallas import tpu_sc as plsc

NC = 2
NS = 16
NW = NC * NS
CHUNK = 128
REAL = 125


def _mesh():
    return plsc.VectorSubcoreMesh(
        core_axis_name="c", subcore_axis_name="s",
        num_cores=NC, num_subcores=NS)


_SC_PARAMS = pltpu.CompilerParams(use_tc_tiling_on_sc=False,
                                  needs_layout_passes=False)


def _rsqrt16(x):
    i = plsc.bitcast(x, jnp.int32)
    y = plsc.bitcast(jnp.int32(0x5F3759DF) - (i >> 1), jnp.float32)
    for _ in range(3):
        y = y * (1.5 - 0.5 * x * y * y)
    return y


def _make_deg(E, n_pad):
    nit = E // NS // REAL
    nit_w = E // NW // REAL
    rps = n_pad // NS
    rph = n_pad // NW

    @functools.partial(
        pl.kernel,
        out_type=jax.ShapeDtypeStruct((n_pad, 16), jnp.float32),
        mesh=_mesh(),
        scratch_types=[
            pltpu.VMEM((nit, CHUNK), jnp.int32),
            pltpu.VMEM((-(-CHUNK // 16) * 16,), jnp.float32),
            pltpu.VMEM((rps,), jnp.float32),
            pltpu.VMEM((rph,), jnp.float32),
            pltpu.VMEM((rph, 16), jnp.float32),
            pltpu.VMEM_SHARED((n_pad,), jnp.float32),
            pltpu.SemaphoreType.DMA,
        ],
        compiler_params=_SC_PARAMS,
    )
    def deg_kernel(e4_hbm, out_hbm, didx, ones_v, stage, ybuf, rows_out,
                   acc, sem):
        cid = lax.axis_index("c")
        sid = lax.axis_index("s")

        def seto(j, carry):
            ones_v[pl.ds(j * 16, 16)] = jnp.ones((16,), jnp.float32)
            return carry
        lax.fori_loop(0, -(-CHUNK // 16), seto, None)

        def setz(j, carry):
            stage[pl.ds(j * 16, 16)] = jnp.zeros((16,), jnp.float32)
            return carry
        lax.fori_loop(0, rps // 16, setz, None)
        pltpu.sync_copy(stage, acc.at[pl.ds(sid * rps, rps)])
        for b in range(NC):
            pltpu.sync_copy(e4_hbm.at[1, NC * sid + b],
                            didx.at[pl.ds(b * nit_w, nit_w)])
        plsc.subcore_barrier()

        def step(i, carry):
            pltpu.async_copy(ones_v, acc.at[didx.at[i]], sem, add=True)
            return carry
        lax.fori_loop(0, nit, step, None)

        def drain(i, carry):
            pltpu.make_async_copy(ones_v, acc.at[didx.at[i]], sem).wait()
            return carry
        lax.fori_loop(0, nit, drain, None)
        plsc.subcore_barrier()

        nbase = cid * (n_pad // NC) + sid * rph
        pltpu.sync_copy(acc.at[pl.ds(nbase, rph)], ybuf)

        def rs(g, carry):
            v = ybuf[pl.ds(g * 16, 16)]
            ybuf[pl.ds(g * 16, 16)] = _rsqrt16(v + 1.0)
            return carry
        lax.fori_loop(0, rph // 16, rs, None)

        def splat(g, carry):
            v = ybuf[pl.ds(g * 16, 16)]
            for j in range(16):
                rows_out[g * 16 + j, :] = jnp.full((16,), v[j], jnp.float32)
            return carry
        lax.fori_loop(0, rph // 16, splat, None)
        pltpu.sync_copy(rows_out, out_hbm.at[pl.ds(nbase, rph)])

    return deg_kernel


def _make_agg(N, E, n_pad, D):
    nit = E // NW // REAL
    rps = n_pad // NS

    NB = 8
    AHEAD = 4
    assert nit % NB == 0 and nit >= 2 * NB

    @functools.partial(
        pl.kernel,
        out_type=jax.ShapeDtypeStruct((NC, n_pad, D), jnp.float32),
        mesh=_mesh(),
        scratch_types=[
            pltpu.VMEM((nit, CHUNK), jnp.int32),
            pltpu.VMEM((nit, CHUNK), jnp.int32),
            pltpu.VMEM((NB, CHUNK, D), jnp.float32),
            pltpu.VMEM((rps, D), jnp.float32),
            pltpu.VMEM_SHARED((n_pad, D), jnp.float32),
            pltpu.VMEM_SHARED((n_pad, D), jnp.float32),
            [pltpu.SemaphoreType.DMA] * NB,
            [pltpu.SemaphoreType.DMA] * NB,
        ],
        compiler_params=_SC_PARAMS,
    )
    def agg_kernel(table_hbm, e4_hbm, out_hbm,
                   sidx, didx, rows, stage, acc, table_sh, gsems, ssems):
        cid = lax.axis_index("c")
        sid = lax.axis_index("s")
        wid = sid * NC + cid

        def setz(j, carry):
            stage[j, :] = jnp.zeros((D,), jnp.float32)
            return carry
        lax.fori_loop(0, rps, setz, None)
        pltpu.sync_copy(stage, acc.at[pl.ds(sid * rps, rps)])

        pltpu.sync_copy(e4_hbm.at[0, wid], sidx)
        pltpu.sync_copy(e4_hbm.at[1, wid], didx)
        pltpu.sync_copy(table_hbm.at[pl.ds(sid * rps, rps)],
                        table_sh.at[pl.ds(sid * rps, rps)])
        plsc.subcore_barrier()

        def _tbl(b):
            return table_sh if b % 2 == 0 else table_hbm

        def gstart(i, b):
            pltpu.async_copy(_tbl(b).at[sidx.at[i]], rows.at[b], gsems[b])

        def gwait(i, b):
            pltpu.make_async_copy(_tbl(b).at[sidx.at[i]], rows.at[b],
                                  gsems[b]).wait()

        def sstart(i, b):
            pltpu.async_copy(rows.at[b], acc.at[didx.at[i]], ssems[b],
                             add=True)

        def swait(i, b):
            pltpu.make_async_copy(rows.at[b], acc.at[didx.at[i]],
                                  ssems[b]).wait()

        for b in range(AHEAD):
            gstart(b, b)

        def round_(k, carry):
            base = NB * k
            for b in range(NB):
                i = base + b
                gwait(i, b)
                sstart(i, b)
                j = i + AHEAD
                bj = (b + AHEAD) % NB

                @pl.when(j < nit)
                def _():
                    @pl.when(j >= NB)
                    def _():
                        swait(j - NB, bj)
                    gstart(j, bj)
            return carry
        lax.fori_loop(0, nit // NB, round_, None)

        for w in range(nit - AHEAD, nit):
            swait(w, w % NB)
        plsc.subcore_barrier()

        pltpu.sync_copy(acc.at[pl.ds(sid * rps, rps)], stage)
        pltpu.sync_copy(stage, out_hbm.at[cid, pl.ds(sid * rps, rps)])

    return agg_kernel


def _mm_pack(x, W, n_pad):
    N, _ = x.shape
    M = W.shape[1]
    P8 = n_pad // 8

    def body(x_ref, w_ref, o_ref):
        w = w_ref[...]
        parts = []
        for i in range(8):
            r0 = P8 * i
            nrows = min(P8, N - r0)
            h = jnp.dot(x_ref[pl.ds(r0, nrows), :], w,
                        preferred_element_type=jnp.float32)
            if nrows < P8:
                h = jnp.concatenate(
                    [h, jnp.zeros((P8 - nrows, M), jnp.float32)], axis=0)
            parts.append(h)
        o_ref[...] = jnp.concatenate(parts, axis=1)

    return pl.pallas_call(
        body, out_shape=jax.ShapeDtypeStruct((P8, 8 * M), jnp.float32),
    )(x, W)


def _scale_mul(hp, dvp):
    def body(h_ref, dv_ref, o_ref):
        o_ref[...] = h_ref[...] * dv_ref[...]

    return pl.pallas_call(
        body, out_shape=jax.ShapeDtypeStruct(hp.shape, jnp.float32),
    )(hp, dvp)


def _mid_layer(pp, hs1p, dvp, b1_tiled, N):
    P8, L = hs1p.shape
    D = L // 8
    g0 = N - 7 * P8

    def body(p_ref, h_ref, dv_ref, b_ref, o_ref):
        dinv = dv_ref[...]
        s = p_ref[0] + p_ref[1] + h_ref[...]
        t = jnp.maximum(s * dinv + b_ref[...], 0.0)
        o_ref[...] = t * dinv
        o_ref[pl.ds(g0, P8 - g0), (L - D):] = jnp.zeros(
            (P8 - g0, D), jnp.float32)

    return pl.pallas_call(
        body, out_shape=jax.ShapeDtypeStruct((P8, L), jnp.float32),
    )(pp, hs1p, dvp, b1_tiled)


def _out_layer(pp, hs2p, dvp, W2, b2_row, N):
    P8, L = hs2p.shape
    D = L // 8
    M = W2.shape[1]

    def body(p_ref, h_ref, dv_ref, w_ref, b_ref, o_ref):
        a = (p_ref[0] + p_ref[1] + h_ref[...]) * dv_ref[...]
        w = w_ref[...]
        b = b_ref[...]
        for i in range(8):
            r0 = P8 * i
            nrows = min(P8, N - r0)
            ai = a[:nrows, D * i:D * (i + 1)]
            o_ref[pl.ds(r0, nrows), :] = jnp.dot(
                ai, w, preferred_element_type=jnp.float32) + b

    return pl.pallas_call(
        body, out_shape=jax.ShapeDtypeStruct((N, M), jnp.float32),
    )(pp, hs2p, dvp, W2, b2_row)


def kernel(x, edge_index, W1, b1, W2, b2):
    N, _ = x.shape
    E = edge_index.shape[1]
    D_H = W1.shape[1]
    D_OUT = W2.shape[1]
    n_pad = -(-N // (NS * 16)) * (NS * 16)

    P8 = n_pad // 8
    nwin = E // REAL
    g0 = N - 7 * P8
    ngh = P8 - g0
    er = edge_index.astype(jnp.int32)
    er = ((er % P8) * 8 + er // P8).reshape(2, nwin, REAL)
    w = jnp.arange(nwin, dtype=jnp.int32)
    pads = 8 * (g0 + ((w[:, None] * (CHUNK - REAL)
                       + jnp.arange(CHUNK - REAL, dtype=jnp.int32)) % ngh)) + 7
    pads = jnp.broadcast_to(pads[None], (2, nwin, CHUNK - REAL))
    e4 = jnp.concatenate([er, pads], axis=2).reshape(
        2, NW, nwin // NW, CHUNK)

    dinv16 = _make_deg(E, n_pad)(e4)
    dvp = dinv16.reshape(P8, 8 * 16)

    agg = _make_agg(N, E, n_pad, D_H)
    b1t = jnp.tile(b1, 8).reshape(1, 8 * D_H)

    hp1 = _mm_pack(x, W1, n_pad)
    hs1p = _scale_mul(hp1, dvp)
    p1p = agg(hs1p.reshape(n_pad, D_H), e4).reshape(NC, P8, 8 * D_H)
    hs2p = _mid_layer(p1p, hs1p, dvp, b1t, N)
    p2p = agg(hs2p.reshape(n_pad, D_H), e4).reshape(NC, P8, 8 * D_H)
    return _out_layer(p2p, hs2p, dvp, W2, b2.reshape(1, D_OUT), N)


# speedup vs baseline: 1.1692x; 1.1692x over previous
import functools

import jax
import jax.numpy as jnp
from jax import lax
from jax.experimental import pallas as pl
from jax.experimental.pallas import tpu as pltpu
from jax.experimental.pallas import tpu_sc as plsc

NC = 2
NS = 16
NW = NC * NS
CHUNK = 128
REAL = 125


def _mesh():
    return plsc.VectorSubcoreMesh(
        core_axis_name="c", subcore_axis_name="s",
        num_cores=NC, num_subcores=NS)


_SC_PARAMS = pltpu.CompilerParams(use_tc_tiling_on_sc=False,
                                  needs_layout_passes=False)


def _rsqrt16(x):
    i = plsc.bitcast(x, jnp.int32)
    y = plsc.bitcast(jnp.int32(0x5F3759DF) - (i >> 1), jnp.float32)
    for _ in range(3):
        y = y * (1.5 - 0.5 * x * y * y)
    return y


def _make_deg(E, n_pad):
    nit = E // NS // REAL
    nit_w = E // NW // REAL
    rps = n_pad // NS
    rph = n_pad // NW

    @functools.partial(
        pl.kernel,
        out_type=jax.ShapeDtypeStruct((n_pad, 16), jnp.float32),
        mesh=_mesh(),
        scratch_types=[
            pltpu.VMEM((nit, CHUNK), jnp.int32),
            pltpu.VMEM((-(-CHUNK // 16) * 16,), jnp.float32),
            pltpu.VMEM((rps,), jnp.float32),
            pltpu.VMEM((rph,), jnp.float32),
            pltpu.VMEM((rph, 16), jnp.float32),
            pltpu.VMEM_SHARED((n_pad,), jnp.float32),
            pltpu.SemaphoreType.DMA,
        ],
        compiler_params=_SC_PARAMS,
    )
    def deg_kernel(e4_hbm, out_hbm, didx, ones_v, stage, ybuf, rows_out,
                   acc, sem):
        cid = lax.axis_index("c")
        sid = lax.axis_index("s")

        def seto(j, carry):
            ones_v[pl.ds(j * 16, 16)] = jnp.ones((16,), jnp.float32)
            return carry
        lax.fori_loop(0, -(-CHUNK // 16), seto, None)

        def setz(j, carry):
            stage[pl.ds(j * 16, 16)] = jnp.zeros((16,), jnp.float32)
            return carry
        lax.fori_loop(0, rps // 16, setz, None)
        pltpu.sync_copy(stage, acc.at[pl.ds(sid * rps, rps)])
        for b in range(NC):
            pltpu.sync_copy(e4_hbm.at[1, NC * sid + b],
                            didx.at[pl.ds(b * nit_w, nit_w)])
        plsc.subcore_barrier()

        def step(i, carry):
            pltpu.async_copy(ones_v, acc.at[didx.at[i]], sem, add=True)
            return carry
        lax.fori_loop(0, nit, step, None)

        def drain(i, carry):
            pltpu.make_async_copy(ones_v, acc.at[didx.at[i]], sem).wait()
            return carry
        lax.fori_loop(0, nit, drain, None)
        plsc.subcore_barrier()

        nbase = cid * (n_pad // NC) + sid * rph
        pltpu.sync_copy(acc.at[pl.ds(nbase, rph)], ybuf)

        def rs(g, carry):
            v = ybuf[pl.ds(g * 16, 16)]
            ybuf[pl.ds(g * 16, 16)] = _rsqrt16(v + 1.0)
            return carry
        lax.fori_loop(0, rph // 16, rs, None)

        def splat(g, carry):
            v = ybuf[pl.ds(g * 16, 16)]
            for j in range(16):
                rows_out[g * 16 + j, :] = jnp.full((16,), v[j], jnp.float32)
            return carry
        lax.fori_loop(0, rph // 16, splat, None)
        pltpu.sync_copy(rows_out, out_hbm.at[pl.ds(nbase, rph)])

    return deg_kernel


def _make_agg(N, E, n_pad, D):
    nit = E // NW // REAL
    rps = n_pad // NS

    NB = 8
    AHEAD = 4
    assert nit % NB == 0 and nit >= 2 * NB

    @functools.partial(
        pl.kernel,
        out_type=jax.ShapeDtypeStruct((NC, n_pad, D), jnp.float32),
        mesh=_mesh(),
        scratch_types=[
            pltpu.VMEM((nit, CHUNK), jnp.int32),
            pltpu.VMEM((nit, CHUNK), jnp.int32),
            pltpu.VMEM((NB, CHUNK, D), jnp.float32),
            pltpu.VMEM((rps, D), jnp.float32),
            pltpu.VMEM_SHARED((n_pad, D), jnp.float32),
            pltpu.VMEM_SHARED((n_pad, D), jnp.float32),
            [pltpu.SemaphoreType.DMA] * NB,
            [pltpu.SemaphoreType.DMA] * NB,
        ],
        compiler_params=_SC_PARAMS,
    )
    def agg_kernel(table_hbm, e4_hbm, out_hbm,
                   sidx, didx, rows, stage, acc, table_sh, gsems, ssems):
        cid = lax.axis_index("c")
        sid = lax.axis_index("s")
        wid = sid * NC + cid
        rsl = pl.ds(sid * rps, rps)

        pltpu.async_copy(e4_hbm.at[0, wid], sidx, gsems[0])
        pltpu.async_copy(e4_hbm.at[1, wid], didx, gsems[1])
        pltpu.async_copy(table_hbm.at[rsl], table_sh.at[rsl], gsems[2])

        @pl.when(cid == 0)
        def _():
            pltpu.async_copy(table_hbm.at[rsl], acc.at[rsl], gsems[3])
            pltpu.make_async_copy(table_hbm.at[rsl], acc.at[rsl],
                                  gsems[3]).wait()

        @pl.when(cid == 1)
        def _():
            def setz(j, carry):
                stage[j, :] = jnp.zeros((D,), jnp.float32)
                return carry
            lax.fori_loop(0, rps, setz, None)
            pltpu.sync_copy(stage, acc.at[rsl])

        pltpu.make_async_copy(e4_hbm.at[0, wid], sidx, gsems[0]).wait()
        pltpu.make_async_copy(e4_hbm.at[1, wid], didx, gsems[1]).wait()
        pltpu.make_async_copy(table_hbm.at[rsl], table_sh.at[rsl],
                              gsems[2]).wait()
        plsc.subcore_barrier()

        def gstart(i, b):
            pltpu.async_copy(table_sh.at[sidx.at[i]], rows.at[b], gsems[b])

        def gwait(i, b):
            pltpu.make_async_copy(table_sh.at[sidx.at[i]], rows.at[b],
                                  gsems[b]).wait()

        def sstart(i, b):
            pltpu.async_copy(rows.at[b], acc.at[didx.at[i]], ssems[b],
                             add=True)

        def swait(i, b):
            pltpu.make_async_copy(rows.at[b], acc.at[didx.at[i]],
                                  ssems[b]).wait()

        for b in range(AHEAD):
            gstart(b, b)

        def round_(k, carry):
            base = NB * k
            for b in range(NB):
                i = base + b
                gwait(i, b)
                sstart(i, b)
                j = i + AHEAD
                bj = (b + AHEAD) % NB

                @pl.when(j < nit)
                def _():
                    @pl.when(j >= NB)
                    def _():
                        swait(j - NB, bj)
                    gstart(j, bj)
            return carry
        lax.fori_loop(0, nit // NB, round_, None)

        for w in range(nit - AHEAD, nit):
            swait(w, w % NB)
        plsc.subcore_barrier()

        pltpu.sync_copy(acc.at[rsl], stage)
        pltpu.sync_copy(stage, out_hbm.at[cid, rsl])

    return agg_kernel


def _mm_pack(x, W, n_pad):
    N, _ = x.shape
    M = W.shape[1]
    P8 = n_pad // 8

    def body(x_ref, w_ref, o_ref):
        w = w_ref[...]
        parts = []
        for i in range(8):
            r0 = P8 * i
            nrows = min(P8, N - r0)
            h = jnp.dot(x_ref[pl.ds(r0, nrows), :], w,
                        preferred_element_type=jnp.float32)
            if nrows < P8:
                h = jnp.concatenate(
                    [h, jnp.zeros((P8 - nrows, M), jnp.float32)], axis=0)
            parts.append(h)
        o_ref[...] = jnp.concatenate(parts, axis=1)

    return pl.pallas_call(
        body, out_shape=jax.ShapeDtypeStruct((P8, 8 * M), jnp.float32),
    )(x, W)


def _scale_mul(hp, dvp):
    def body(h_ref, dv_ref, o_ref):
        o_ref[...] = h_ref[...] * dv_ref[...]

    return pl.pallas_call(
        body, out_shape=jax.ShapeDtypeStruct(hp.shape, jnp.float32),
    )(hp, dvp)


def _mid_layer(pp, dvp, b1_tiled, N):
    _, P8, L = pp.shape
    D = L // 8
    g0 = N - 7 * P8

    def body(p_ref, dv_ref, b_ref, o_ref):
        dinv = dv_ref[...]
        s = p_ref[0] + p_ref[1]
        t = jnp.maximum(s * dinv + b_ref[...], 0.0)
        o_ref[...] = t * dinv
        o_ref[pl.ds(g0, P8 - g0), (L - D):] = jnp.zeros(
            (P8 - g0, D), jnp.float32)

    return pl.pallas_call(
        body, out_shape=jax.ShapeDtypeStruct((P8, L), jnp.float32),
    )(pp, dvp, b1_tiled)


def _out_layer(pp, dvp, W2, b2_row, N):
    _, P8, L = pp.shape
    D = L // 8
    M = W2.shape[1]

    def body(p_ref, dv_ref, w_ref, b_ref, o_ref):
        a = (p_ref[0] + p_ref[1]) * dv_ref[...]
        w = w_ref[...]
        b = b_ref[...]
        for i in range(8):
            r0 = P8 * i
            nrows = min(P8, N - r0)
            ai = a[:nrows, D * i:D * (i + 1)]
            o_ref[pl.ds(r0, nrows), :] = jnp.dot(
                ai, w, preferred_element_type=jnp.float32) + b

    return pl.pallas_call(
        body, out_shape=jax.ShapeDtypeStruct((N, M), jnp.float32),
    )(pp, dvp, W2, b2_row)


def kernel(x, edge_index, W1, b1, W2, b2):
    N, _ = x.shape
    E = edge_index.shape[1]
    D_H = W1.shape[1]
    D_OUT = W2.shape[1]
    n_pad = -(-N // (NS * 16)) * (NS * 16)

    P8 = n_pad // 8
    nwin = E // REAL
    g0 = N - 7 * P8
    ngh = P8 - g0
    er = edge_index.astype(jnp.int32)
    er = ((er % P8) * 8 + er // P8).reshape(2, nwin, REAL)
    w = jnp.arange(nwin, dtype=jnp.int32)
    pads = 8 * (g0 + ((w[:, None] * (CHUNK - REAL)
                       + jnp.arange(CHUNK - REAL, dtype=jnp.int32)) % ngh)) + 7
    pads = jnp.broadcast_to(pads[None], (2, nwin, CHUNK - REAL))
    e4 = jnp.concatenate([er, pads], axis=2).reshape(
        2, NW, nwin // NW, CHUNK)

    dinv16 = _make_deg(E, n_pad)(e4)
    dvp = dinv16.reshape(P8, 8 * 16)

    agg = _make_agg(N, E, n_pad, D_H)
    b1t = jnp.tile(b1, 8).reshape(1, 8 * D_H)

    hp1 = _mm_pack(x, W1, n_pad)
    hs1p = _scale_mul(hp1, dvp)
    p1p = agg(hs1p.reshape(n_pad, D_H), e4).reshape(NC, P8, 8 * D_H)
    hs2p = _mid_layer(p1p, dvp, b1t, N)
    p2p = agg(hs2p.reshape(n_pad, D_H), e4).reshape(NC, P8, 8 * D_H)
    return _out_layer(p2p, dvp, W2, b2.reshape(1, D_OUT), N)
